# baseline (speedup 1.0000x reference)
import functools

import jax
import jax.numpy as jnp
import numpy as np
from jax import lax
from jax.experimental import pallas as pl
from jax.experimental.pallas import tpu as pltpu
from jax.experimental.pallas import tpu_sc as plsc

N = 10000
E = 160000
D = 128
C = 64

NW = 32
CH = 128
EPW = 5120
EPAD = NW * EPW
NCH = EPW // CH
NP = 10112
RPT = NP // 16

_SQRT3C_INV = float(1.0 / np.sqrt(3 * C))
_EPS = 1e-5

f32 = jnp.float32


_HI = np.uint32(0xFFFF0000)


def _node_proj_body(x_ref, w_ref, b_ref, t_ref):
    t = jnp.dot(x_ref[...], w_ref[...], preferred_element_type=f32) + b_ref[...]
    a = lax.bitcast_convert_type(t[:, :96], jnp.uint32) & _HI
    b = lax.bitcast_convert_type(t[:, 96:], jnp.uint32) >> 16
    packed = lax.bitcast_convert_type(a | b, f32)
    t_ref[pl.ds(0, N), :] = jnp.concatenate(
        [packed, jnp.zeros((N, 32), f32)], axis=1)


def _node_proj(x, wqkv_t, bqkv):
    return pl.pallas_call(
        _node_proj_body,
        out_shape=jax.ShapeDtypeStruct((NP, 2 * C), f32),
    )(x, wqkv_t, bqkv)


K0 = 24
K1 = 56


def _gather_body(tab_hbm, dst2d_hbm, src2d_hbm, g1_hbm, g2_hbm,
                 idxd_v, idxs_v, rows1_v, rows2_v, semd0, semd1, sems0):
    c = lax.axis_index("c")
    s = lax.axis_index("s")
    nch = jnp.where(c == 0, K0, K1)
    base_rows = jnp.where(c == 0, s * K0, 16 * K0 + s * K1)

    @pl.when(c == 0)
    def _():
        pltpu.sync_copy(dst2d_hbm.at[pl.ds(s * K0, K0)],
                        idxd_v.at[pl.ds(0, K0)])
        pltpu.sync_copy(src2d_hbm.at[pl.ds(s * K0, K0)],
                        idxs_v.at[pl.ds(0, K0)])

    @pl.when(c == 1)
    def _():
        pltpu.sync_copy(dst2d_hbm.at[pl.ds(16 * K0 + s * K1, K1)],
                        idxd_v.at[pl.ds(0, K1)])
        pltpu.sync_copy(src2d_hbm.at[pl.ds(16 * K0 + s * K1, K1)],
                        idxs_v.at[pl.ds(0, K1)])
    def issue_d(j, b, sem):
        pltpu.async_copy(tab_hbm.at[idxd_v.at[j]], rows1_v.at[b], sem)

    def issue_s(j):
        pltpu.async_copy(tab_hbm.at[idxs_v.at[j]], rows2_v, sems0)

    def wait_d(j, b, sem):
        pltpu.make_async_copy(tab_hbm.at[idxd_v.at[j]], rows1_v.at[b],
                              sem).wait()

    def wait_s(j):
        pltpu.make_async_copy(tab_hbm.at[idxs_v.at[j]], rows2_v, sems0).wait()

    issue_d(0, 0, semd0)

    @pl.when(1 < nch)
    def _():
        issue_d(1, 1, semd1)

    issue_s(0)

    def body(p, carry):
        j0 = 2 * p

        wait_d(j0, 0, semd0)
        pltpu.sync_copy(rows1_v.at[0], g1_hbm.at[pl.ds((base_rows + j0) * CH, CH)])

        @pl.when(j0 + 2 < nch)
        def _():
            issue_d(j0 + 2, 0, semd0)

        wait_s(j0)
        pltpu.sync_copy(rows2_v, g2_hbm.at[pl.ds((base_rows + j0) * CH, CH)])

        @pl.when(j0 + 1 < nch)
        def _():
            issue_s(j0 + 1)

        @pl.when(j0 + 1 < nch)
        def _():
            wait_d(j0 + 1, 1, semd1)
            pltpu.sync_copy(rows1_v.at[1],
                            g1_hbm.at[pl.ds((base_rows + j0 + 1) * CH, CH)])

            @pl.when(j0 + 3 < nch)
            def _():
                issue_d(j0 + 3, 1, semd1)

            wait_s(j0 + 1)
            pltpu.sync_copy(rows2_v,
                            g2_hbm.at[pl.ds((base_rows + j0 + 1) * CH, CH)])

            @pl.when(j0 + 2 < nch)
            def _():
                issue_s(j0 + 2)

        return carry

    lax.fori_loop(0, (nch + 1) // 2, body, 0)


def _gather(tab, dst2d, src2d):
    mesh = plsc.VectorSubcoreMesh(core_axis_name="c", subcore_axis_name="s")
    eshape = jax.ShapeDtypeStruct((EPAD, 2 * C), f32)
    return pl.kernel(
        _gather_body,
        out_type=[eshape, eshape],
        mesh=mesh,
        scratch_types=[
            pltpu.VMEM((max(K0, K1), CH), jnp.int32),
            pltpu.VMEM((max(K0, K1), CH), jnp.int32),
            pltpu.VMEM((2, CH, 2 * C), f32),
            pltpu.VMEM((CH, 2 * C), f32),
            pltpu.SemaphoreType.DMA,
            pltpu.SemaphoreType.DMA,
            pltpu.SemaphoreType.DMA,
        ],
    )(tab, dst2d, src2d)


def _unpack_qkv(p):
    pb = lax.bitcast_convert_type(p, jnp.uint32)
    a = lax.bitcast_convert_type(pb & _HI, f32)
    b = lax.bitcast_convert_type(pb << 16, f32)
    q = a[:, :C]
    k = jnp.concatenate([a[:, C:96], b[:, :32]], axis=1)
    v = b[:, 32:96]
    return q, k, v


def _edge_body(g1_ref, g2_ref, ea_ref, wet_ref, wmut_ref, bmu_ref,
               lnag_ref, lnab_ref, wmlt_ref, bml_ref, lnmg_ref, lnmb_ref,
               m_ref):
    q_i, k_i, v_i = _unpack_qkv(g1_ref[...])
    _, k_j, v_j = _unpack_qkv(g2_ref[...])
    e = jnp.dot(ea_ref[...], wet_ref[...], preferred_element_type=f32)

    alpha = jnp.concatenate([q_i * k_i, q_i * k_j, q_i * e], axis=1)
    alpha = alpha * _SQRT3C_INV
    mu = jnp.mean(alpha, axis=1, keepdims=True)
    d = alpha - mu
    var = jnp.mean(d * d, axis=1, keepdims=True)
    ln = d * lax.rsqrt(var + _EPS) * lnag_ref[...] + lnab_ref[...]
    gate = jax.nn.sigmoid(ln)

    wmut = wmut_ref[...]
    u = (jnp.dot(v_i, wmut[:C], preferred_element_type=f32)
         + jnp.dot(v_j, wmut[C:2 * C], preferred_element_type=f32)
         + jnp.dot(e, wmut[2 * C:], preferred_element_type=f32)
         + bmu_ref[...])
    m = jnp.dot(u * gate, wmlt_ref[...], preferred_element_type=f32) + bml_ref[...]
    mmu = jnp.mean(m, axis=1, keepdims=True)
    md = m - mmu
    mvar = jnp.mean(md * md, axis=1, keepdims=True)
    mm = md * lax.rsqrt(mvar + _EPS) * lnmg_ref[...] + lnmb_ref[...]
    m_ref[...] = jnp.concatenate([mm, jnp.zeros_like(mm)], axis=1)


def _edge_math(g1, g2, ea, wet, wmut, bmu, lnag, lnab, wmlt, bml,
               lnmg, lnmb):
    BE = 2000
    grid = E // BE
    full = lambda r, c_: pl.BlockSpec((r, c_), lambda i: (0, 0))
    eblk = pl.BlockSpec((BE, 2 * C), lambda i: (i, 0))
    return pl.pallas_call(
        _edge_body,
        grid=(grid,),
        in_specs=[
            eblk,
            eblk,
            pl.BlockSpec((BE, 16), lambda i: (i, 0)),
            full(16, C),
            full(3 * C, 3 * C),
            full(1, 3 * C),
            full(1, 3 * C),
            full(1, 3 * C),
            full(3 * C, C),
            full(1, C),
            full(1, C),
            full(1, C),
        ],
        out_specs=pl.BlockSpec((BE, 2 * C), lambda i: (i, 0)),
        out_shape=jax.ShapeDtypeStruct((EPAD, 2 * C), f32),
    )(g1, g2, ea, wet, wmut, bmu, lnag, lnab, wmlt, bml, lnmg, lnmb)


def _scatter_body(m_hbm, dst2d_hbm, zeros_hbm, out_hbm, mrows_v, idx_v,
                  semm0, semm1, agg_sh):
    c = lax.axis_index("c")
    s = lax.axis_index("s")
    wid = s * 2 + c
    pltpu.sync_copy(zeros_hbm.at[pl.ds(s * RPT, RPT)],
                    agg_sh.at[pl.ds(s * RPT, RPT)])
    plsc.subcore_barrier()

    base_rows = wid * NCH
    pltpu.sync_copy(dst2d_hbm.at[pl.ds(base_rows, NCH)], idx_v)
    semm = (semm0, semm1)

    def load(j, b):
        pltpu.async_copy(m_hbm.at[pl.ds((base_rows + j) * CH, CH)],
                         mrows_v.at[b], semm[b])

    def drain_scatter(j, b):
        pltpu.make_async_copy(m_hbm.at[pl.ds((base_rows + j) * CH, CH)],
                              mrows_v.at[b], semm[b]).wait()
        pltpu.sync_copy(mrows_v.at[b], agg_sh.at[idx_v.at[j]], add=True)

    load(0, 0)

    def body(p, carry):
        j0 = 2 * p
        load(j0 + 1, 1)
        drain_scatter(j0, 0)

        @pl.when(j0 + 2 < NCH)
        def _():
            load(j0 + 2, 0)

        drain_scatter(j0 + 1, 1)
        return carry

    lax.fori_loop(0, NCH // 2, body, 0)
    plsc.subcore_barrier()
    pltpu.sync_copy(agg_sh.at[pl.ds(s * RPT, RPT)],
                    out_hbm.at[c].at[pl.ds(s * RPT, RPT)])


def _scatter(m, dst2d, zeros_np):
    mesh = plsc.VectorSubcoreMesh(core_axis_name="c", subcore_axis_name="s")
    return pl.kernel(
        _scatter_body,
        out_type=jax.ShapeDtypeStruct((2, NP, 2 * C), f32),
        mesh=mesh,
        scratch_types=[
            pltpu.VMEM((2, CH, 2 * C), f32),
            pltpu.VMEM((NCH, CH), jnp.int32),
            pltpu.SemaphoreType.DMA,
            pltpu.SemaphoreType.DMA,
            pltpu.VMEM_SHARED((NP, 2 * C), f32),
        ],
    )(m, dst2d, zeros_np)


def _out_body(agg_ref, x_ref, wcat_ref, bcat_ref, bng_ref, bnb_ref,
              wskip_ref, bskip_ref, out_ref):
    agg = (agg_ref[0, :N, :C] + agg_ref[1, :N, :C])
    o = jnp.dot(agg, wcat_ref[...], preferred_element_type=f32) + bcat_ref[...]
    mu = jnp.mean(o, axis=0, keepdims=True)
    d = o - mu
    var = jnp.mean(d * d, axis=0, keepdims=True)
    o = d * lax.rsqrt(var + _EPS) * bng_ref[...] + bnb_ref[...]
    o = o * jax.nn.sigmoid(o)
    skip = jnp.dot(x_ref[...], wskip_ref[...], preferred_element_type=f32)
    out_ref[...] = o + skip + bskip_ref[...]


def _node_out(agg2, x, wcat_t, bcat, bng, bnb, wskip_t, bskip):
    return pl.pallas_call(
        _out_body,
        out_shape=jax.ShapeDtypeStruct((N, C), f32),
    )(agg2, x, wcat_t, bcat, bng, bnb, wskip_t, bskip)


def kernel(x, edge_index, edge_attr, Wq, bq, Wk, bk, Wv, bv, We, W_mu, b_mu,
           ln_a_g, ln_a_b, W_ml, b_ml, ln_m_g, ln_m_b, W_cat, b_cat,
           bn_g, bn_b, W_skip, b_skip):
    src = edge_index[0].astype(jnp.int32)
    dst = edge_index[1].astype(jnp.int32)
    pad_e = EPAD - E
    src_pad = jnp.concatenate([src, jnp.full((pad_e,), N, jnp.int32)])
    dst_pad = jnp.concatenate([dst, jnp.full((pad_e,), N, jnp.int32)])
    dst2d = dst_pad.reshape(EPAD // CH, CH)
    src2d = src_pad.reshape(EPAD // CH, CH)

    wqkv_t = jnp.concatenate([Wq, Wk, Wv], axis=0).T
    bqkv = jnp.concatenate([bq, bk, bv]).reshape(1, 3 * C)

    tab = _node_proj(x, wqkv_t, bqkv)
    g1, g2 = _gather(tab, dst2d, src2d)

    m = _edge_math(
        g1, g2, edge_attr, We.T, W_mu.T, b_mu.reshape(1, -1),
        ln_a_g.reshape(1, -1), ln_a_b.reshape(1, -1), W_ml.T,
        b_ml.reshape(1, -1), ln_m_g.reshape(1, -1), ln_m_b.reshape(1, -1))

    zeros_np = jnp.zeros((NP, 2 * C), f32)
    agg2 = _scatter(m, dst2d, zeros_np)

    return _node_out(agg2, x, W_cat.T, b_cat.reshape(1, -1),
                     bn_g.reshape(1, -1), bn_b.reshape(1, -1), W_skip.T,
                     b_skip.reshape(1, -1))


# speedup vs baseline: 1.0214x; 1.0214x over previous
import functools

import jax
import jax.numpy as jnp
import numpy as np
from jax import lax
from jax.experimental import pallas as pl
from jax.experimental.pallas import tpu as pltpu
from jax.experimental.pallas import tpu_sc as plsc

N = 10000
E = 160000
D = 128
C = 64

NW = 32
CH = 128
EPW = 5120
EPAD = NW * EPW
NCH = EPW // CH
NP = 10112
RPT = NP // 16

_SQRT3C_INV = float(1.0 / np.sqrt(3 * C))
_EPS = 1e-5

f32 = jnp.float32


_HI = np.uint32(0xFFFF0000)


def _node_proj_body(x_ref, w_ref, b_ref, t_ref):
    t = jnp.dot(x_ref[...], w_ref[...], preferred_element_type=f32) + b_ref[...]
    a = lax.bitcast_convert_type(t[:, :96], jnp.uint32) & _HI
    b = lax.bitcast_convert_type(t[:, 96:], jnp.uint32) >> 16
    packed = lax.bitcast_convert_type(a | b, f32)
    t_ref[pl.ds(0, N), :] = jnp.concatenate(
        [packed, jnp.zeros((N, 32), f32)], axis=1)


def _node_proj(x, wqkv_t, bqkv):
    return pl.pallas_call(
        _node_proj_body,
        out_shape=jax.ShapeDtypeStruct((NP, 2 * C), f32),
    )(x, wqkv_t, bqkv)


K0 = 56
K1 = 24


def _gather_body(tab_hbm, dst2d_hbm, src2d_hbm, g1_hbm, g2_hbm,
                 idxd_v, idxs_v, rows1_v, rows2_v, semd0, semd1, sems0):
    c = lax.axis_index("c")
    s = lax.axis_index("s")
    nch = jnp.where(c == 0, K0, K1)
    base_rows = jnp.where(c == 0, s * K0, 16 * K0 + s * K1)

    @pl.when(c == 0)
    def _():
        pltpu.sync_copy(dst2d_hbm.at[pl.ds(s * K0, K0)],
                        idxd_v.at[pl.ds(0, K0)])
        pltpu.sync_copy(src2d_hbm.at[pl.ds(s * K0, K0)],
                        idxs_v.at[pl.ds(0, K0)])

    @pl.when(c == 1)
    def _():
        pltpu.sync_copy(dst2d_hbm.at[pl.ds(16 * K0 + s * K1, K1)],
                        idxd_v.at[pl.ds(0, K1)])
        pltpu.sync_copy(src2d_hbm.at[pl.ds(16 * K0 + s * K1, K1)],
                        idxs_v.at[pl.ds(0, K1)])
    def issue_d(j, b, sem):
        pltpu.async_copy(tab_hbm.at[idxd_v.at[j]], rows1_v.at[b], sem)

    def issue_s(j):
        pltpu.async_copy(tab_hbm.at[idxs_v.at[j]], rows2_v, sems0)

    def wait_d(j, b, sem):
        pltpu.make_async_copy(tab_hbm.at[idxd_v.at[j]], rows1_v.at[b],
                              sem).wait()

    def wait_s(j):
        pltpu.make_async_copy(tab_hbm.at[idxs_v.at[j]], rows2_v, sems0).wait()

    issue_d(0, 0, semd0)

    @pl.when(1 < nch)
    def _():
        issue_d(1, 1, semd1)

    issue_s(0)

    def body(p, carry):
        j0 = 2 * p

        wait_d(j0, 0, semd0)
        pltpu.sync_copy(rows1_v.at[0], g1_hbm.at[pl.ds((base_rows + j0) * CH, CH)])

        @pl.when(j0 + 2 < nch)
        def _():
            issue_d(j0 + 2, 0, semd0)

        wait_s(j0)
        pltpu.sync_copy(rows2_v, g2_hbm.at[pl.ds((base_rows + j0) * CH, CH)])

        @pl.when(j0 + 1 < nch)
        def _():
            issue_s(j0 + 1)

        @pl.when(j0 + 1 < nch)
        def _():
            wait_d(j0 + 1, 1, semd1)
            pltpu.sync_copy(rows1_v.at[1],
                            g1_hbm.at[pl.ds((base_rows + j0 + 1) * CH, CH)])

            @pl.when(j0 + 3 < nch)
            def _():
                issue_d(j0 + 3, 1, semd1)

            wait_s(j0 + 1)
            pltpu.sync_copy(rows2_v,
                            g2_hbm.at[pl.ds((base_rows + j0 + 1) * CH, CH)])

            @pl.when(j0 + 2 < nch)
            def _():
                issue_s(j0 + 2)

        return carry

    lax.fori_loop(0, (nch + 1) // 2, body, 0)


def _gather(tab, dst2d, src2d):
    mesh = plsc.VectorSubcoreMesh(core_axis_name="c", subcore_axis_name="s")
    eshape = jax.ShapeDtypeStruct((EPAD, 2 * C), f32)
    return pl.kernel(
        _gather_body,
        out_type=[eshape, eshape],
        mesh=mesh,
        scratch_types=[
            pltpu.VMEM((max(K0, K1), CH), jnp.int32),
            pltpu.VMEM((max(K0, K1), CH), jnp.int32),
            pltpu.VMEM((2, CH, 2 * C), f32),
            pltpu.VMEM((CH, 2 * C), f32),
            pltpu.SemaphoreType.DMA,
            pltpu.SemaphoreType.DMA,
            pltpu.SemaphoreType.DMA,
        ],
    )(tab, dst2d, src2d)


def _unpack_qkv(p):
    pb = lax.bitcast_convert_type(p, jnp.uint32)
    a = lax.bitcast_convert_type(pb & _HI, f32)
    b = lax.bitcast_convert_type(pb << 16, f32)
    q = a[:, :C]
    k = jnp.concatenate([a[:, C:96], b[:, :32]], axis=1)
    v = b[:, 32:96]
    return q, k, v


def _edge_body(g1_ref, g2_ref, ea_ref, wet_ref, wmut_ref, bmu_ref,
               lnag_ref, lnab_ref, wmlt_ref, bml_ref, lnmg_ref, lnmb_ref,
               m_ref):
    q_i, k_i, v_i = _unpack_qkv(g1_ref[...])
    _, k_j, v_j = _unpack_qkv(g2_ref[...])
    e = jnp.dot(ea_ref[...], wet_ref[...], preferred_element_type=f32)

    alpha = jnp.concatenate([q_i * k_i, q_i * k_j, q_i * e], axis=1)
    alpha = alpha * _SQRT3C_INV
    mu = jnp.mean(alpha, axis=1, keepdims=True)
    d = alpha - mu
    var = jnp.mean(d * d, axis=1, keepdims=True)
    ln = d * lax.rsqrt(var + _EPS) * lnag_ref[...] + lnab_ref[...]
    gate = jax.nn.sigmoid(ln)

    wmut = wmut_ref[...]
    u = (jnp.dot(v_i, wmut[:C], preferred_element_type=f32)
         + jnp.dot(v_j, wmut[C:2 * C], preferred_element_type=f32)
         + jnp.dot(e, wmut[2 * C:], preferred_element_type=f32)
         + bmu_ref[...])
    m = jnp.dot(u * gate, wmlt_ref[...], preferred_element_type=f32) + bml_ref[...]
    mmu = jnp.mean(m, axis=1, keepdims=True)
    md = m - mmu
    mvar = jnp.mean(md * md, axis=1, keepdims=True)
    mm = md * lax.rsqrt(mvar + _EPS) * lnmg_ref[...] + lnmb_ref[...]
    m_ref[...] = jnp.concatenate([mm, jnp.zeros_like(mm)], axis=1)


def _edge_math(g1, g2, ea, wet, wmut, bmu, lnag, lnab, wmlt, bml,
               lnmg, lnmb):
    BE = 2000
    grid = E // BE
    full = lambda r, c_: pl.BlockSpec((r, c_), lambda i: (0, 0))
    eblk = pl.BlockSpec((BE, 2 * C), lambda i: (i, 0))
    return pl.pallas_call(
        _edge_body,
        grid=(grid,),
        in_specs=[
            eblk,
            eblk,
            pl.BlockSpec((BE, 16), lambda i: (i, 0)),
            full(16, C),
            full(3 * C, 3 * C),
            full(1, 3 * C),
            full(1, 3 * C),
            full(1, 3 * C),
            full(3 * C, C),
            full(1, C),
            full(1, C),
            full(1, C),
        ],
        out_specs=pl.BlockSpec((BE, 2 * C), lambda i: (i, 0)),
        out_shape=jax.ShapeDtypeStruct((EPAD, 2 * C), f32),
    )(g1, g2, ea, wet, wmut, bmu, lnag, lnab, wmlt, bml, lnmg, lnmb)


def _scatter_body(m_hbm, dst2d_hbm, zeros_hbm, out_hbm, mrows_v, idx_v,
                  semm0, semm1, agg_sh):
    c = lax.axis_index("c")
    s = lax.axis_index("s")
    wid = s * 2 + c
    pltpu.sync_copy(zeros_hbm.at[pl.ds(s * RPT, RPT)],
                    agg_sh.at[pl.ds(s * RPT, RPT)])
    plsc.subcore_barrier()

    base_rows = wid * NCH
    pltpu.sync_copy(dst2d_hbm.at[pl.ds(base_rows, NCH)], idx_v)
    semm = (semm0, semm1)

    def load(j, b):
        pltpu.async_copy(m_hbm.at[pl.ds((base_rows + j) * CH, CH)],
                         mrows_v.at[b], semm[b])

    def drain_scatter(j, b):
        pltpu.make_async_copy(m_hbm.at[pl.ds((base_rows + j) * CH, CH)],
                              mrows_v.at[b], semm[b]).wait()
        pltpu.sync_copy(mrows_v.at[b], agg_sh.at[idx_v.at[j]], add=True)

    load(0, 0)

    def body(p, carry):
        j0 = 2 * p
        load(j0 + 1, 1)
        drain_scatter(j0, 0)

        @pl.when(j0 + 2 < NCH)
        def _():
            load(j0 + 2, 0)

        drain_scatter(j0 + 1, 1)
        return carry

    lax.fori_loop(0, NCH // 2, body, 0)
    plsc.subcore_barrier()
    pltpu.sync_copy(agg_sh.at[pl.ds(s * RPT, RPT)],
                    out_hbm.at[c].at[pl.ds(s * RPT, RPT)])


def _scatter(m, dst2d, zeros_np):
    mesh = plsc.VectorSubcoreMesh(core_axis_name="c", subcore_axis_name="s")
    return pl.kernel(
        _scatter_body,
        out_type=jax.ShapeDtypeStruct((2, NP, 2 * C), f32),
        mesh=mesh,
        scratch_types=[
            pltpu.VMEM((2, CH, 2 * C), f32),
            pltpu.VMEM((NCH, CH), jnp.int32),
            pltpu.SemaphoreType.DMA,
            pltpu.SemaphoreType.DMA,
            pltpu.VMEM_SHARED((NP, 2 * C), f32),
        ],
    )(m, dst2d, zeros_np)


def _out_body(agg_ref, x_ref, wcat_ref, bcat_ref, bng_ref, bnb_ref,
              wskip_ref, bskip_ref, out_ref):
    agg = (agg_ref[0, :N, :C] + agg_ref[1, :N, :C])
    o = jnp.dot(agg, wcat_ref[...], preferred_element_type=f32) + bcat_ref[...]
    mu = jnp.mean(o, axis=0, keepdims=True)
    d = o - mu
    var = jnp.mean(d * d, axis=0, keepdims=True)
    o = d * lax.rsqrt(var + _EPS) * bng_ref[...] + bnb_ref[...]
    o = o * jax.nn.sigmoid(o)
    skip = jnp.dot(x_ref[...], wskip_ref[...], preferred_element_type=f32)
    out_ref[...] = o + skip + bskip_ref[...]


def _node_out(agg2, x, wcat_t, bcat, bng, bnb, wskip_t, bskip):
    return pl.pallas_call(
        _out_body,
        out_shape=jax.ShapeDtypeStruct((N, C), f32),
    )(agg2, x, wcat_t, bcat, bng, bnb, wskip_t, bskip)


def kernel(x, edge_index, edge_attr, Wq, bq, Wk, bk, Wv, bv, We, W_mu, b_mu,
           ln_a_g, ln_a_b, W_ml, b_ml, ln_m_g, ln_m_b, W_cat, b_cat,
           bn_g, bn_b, W_skip, b_skip):
    src = edge_index[0].astype(jnp.int32)
    dst = edge_index[1].astype(jnp.int32)
    pad_e = EPAD - E
    src_pad = jnp.concatenate([src, jnp.full((pad_e,), N, jnp.int32)])
    dst_pad = jnp.concatenate([dst, jnp.full((pad_e,), N, jnp.int32)])
    dst2d = dst_pad.reshape(EPAD // CH, CH)
    src2d = src_pad.reshape(EPAD // CH, CH)

    wqkv_t = jnp.concatenate([Wq, Wk, Wv], axis=0).T
    bqkv = jnp.concatenate([bq, bk, bv]).reshape(1, 3 * C)

    tab = _node_proj(x, wqkv_t, bqkv)
    g1, g2 = _gather(tab, dst2d, src2d)

    m = _edge_math(
        g1, g2, edge_attr, We.T, W_mu.T, b_mu.reshape(1, -1),
        ln_a_g.reshape(1, -1), ln_a_b.reshape(1, -1), W_ml.T,
        b_ml.reshape(1, -1), ln_m_g.reshape(1, -1), ln_m_b.reshape(1, -1))

    zeros_np = jnp.zeros((NP, 2 * C), f32)
    agg2 = _scatter(m, dst2d, zeros_np)

    return _node_out(agg2, x, W_cat.T, b_cat.reshape(1, -1),
                     bn_g.reshape(1, -1), bn_b.reshape(1, -1), W_skip.T,
                     b_skip.reshape(1, -1))


# speedup vs baseline: 1.0318x; 1.0102x over previous
import functools

import jax
import jax.numpy as jnp
import numpy as np
from jax import lax
from jax.experimental import pallas as pl
from jax.experimental.pallas import tpu as pltpu
from jax.experimental.pallas import tpu_sc as plsc

N = 10000
E = 160000
D = 128
C = 64

NW = 32
CH = 128
EPW = 5120
EPAD = NW * EPW
NCH = EPW // CH
NP = 10112
RPT = NP // 16

_SQRT3C_INV = float(1.0 / np.sqrt(3 * C))
_EPS = 1e-5

f32 = jnp.float32


_HI = np.uint32(0xFFFF0000)


def _node_proj_body(x_ref, w_ref, b_ref, t_ref):
    t = jnp.dot(x_ref[...], w_ref[...], preferred_element_type=f32) + b_ref[...]
    a = lax.bitcast_convert_type(t[:, :96], jnp.uint32) & _HI
    b = lax.bitcast_convert_type(t[:, 96:], jnp.uint32) >> 16
    packed = lax.bitcast_convert_type(a | b, f32)
    t_ref[pl.ds(0, N), :] = jnp.concatenate(
        [packed, jnp.zeros((N, 32), f32)], axis=1)


def _node_proj(x, wqkv_t, bqkv):
    return pl.pallas_call(
        _node_proj_body,
        out_shape=jax.ShapeDtypeStruct((NP, 2 * C), f32),
    )(x, wqkv_t, bqkv)


K0 = 40
K1 = 40


def _gather_body(tab_hbm, dst2d_hbm, src2d_hbm, g1_hbm, g2_hbm,
                 idxd_v, idxs_v, rows1_v, rows2_v, semd0, semd1, sems0):
    c = lax.axis_index("c")
    s = lax.axis_index("s")
    nch = jnp.where(c == 0, K0, K1)
    base_rows = jnp.where(c == 0, s * K0, 16 * K0 + s * K1)

    @pl.when(c == 0)
    def _():
        pltpu.sync_copy(dst2d_hbm.at[pl.ds(s * K0, K0)],
                        idxd_v.at[pl.ds(0, K0)])
        pltpu.sync_copy(src2d_hbm.at[pl.ds(s * K0, K0)],
                        idxs_v.at[pl.ds(0, K0)])

    @pl.when(c == 1)
    def _():
        pltpu.sync_copy(dst2d_hbm.at[pl.ds(16 * K0 + s * K1, K1)],
                        idxd_v.at[pl.ds(0, K1)])
        pltpu.sync_copy(src2d_hbm.at[pl.ds(16 * K0 + s * K1, K1)],
                        idxs_v.at[pl.ds(0, K1)])
    def issue_d(j, b, sem):
        pltpu.async_copy(tab_hbm.at[idxd_v.at[j]], rows1_v.at[b], sem)

    def issue_s(j):
        pltpu.async_copy(tab_hbm.at[idxs_v.at[j]], rows2_v, sems0)

    def wait_d(j, b, sem):
        pltpu.make_async_copy(tab_hbm.at[idxd_v.at[j]], rows1_v.at[b],
                              sem).wait()

    def wait_s(j):
        pltpu.make_async_copy(tab_hbm.at[idxs_v.at[j]], rows2_v, sems0).wait()

    issue_d(0, 0, semd0)

    @pl.when(1 < nch)
    def _():
        issue_d(1, 1, semd1)

    issue_s(0)

    def body(p, carry):
        j0 = 2 * p

        wait_d(j0, 0, semd0)
        pltpu.sync_copy(rows1_v.at[0], g1_hbm.at[pl.ds((base_rows + j0) * CH, CH)])

        @pl.when(j0 + 2 < nch)
        def _():
            issue_d(j0 + 2, 0, semd0)

        wait_s(j0)
        pltpu.sync_copy(rows2_v, g2_hbm.at[pl.ds((base_rows + j0) * CH, CH)])

        @pl.when(j0 + 1 < nch)
        def _():
            issue_s(j0 + 1)

        @pl.when(j0 + 1 < nch)
        def _():
            wait_d(j0 + 1, 1, semd1)
            pltpu.sync_copy(rows1_v.at[1],
                            g1_hbm.at[pl.ds((base_rows + j0 + 1) * CH, CH)])

            @pl.when(j0 + 3 < nch)
            def _():
                issue_d(j0 + 3, 1, semd1)

            wait_s(j0 + 1)
            pltpu.sync_copy(rows2_v,
                            g2_hbm.at[pl.ds((base_rows + j0 + 1) * CH, CH)])

            @pl.when(j0 + 2 < nch)
            def _():
                issue_s(j0 + 2)

        return carry

    lax.fori_loop(0, (nch + 1) // 2, body, 0)


def _gather(tab, dst2d, src2d):
    mesh = plsc.VectorSubcoreMesh(core_axis_name="c", subcore_axis_name="s")
    eshape = jax.ShapeDtypeStruct((EPAD, 2 * C), f32)
    return pl.kernel(
        _gather_body,
        out_type=[eshape, eshape],
        mesh=mesh,
        scratch_types=[
            pltpu.VMEM((max(K0, K1), CH), jnp.int32),
            pltpu.VMEM((max(K0, K1), CH), jnp.int32),
            pltpu.VMEM((2, CH, 2 * C), f32),
            pltpu.VMEM((CH, 2 * C), f32),
            pltpu.SemaphoreType.DMA,
            pltpu.SemaphoreType.DMA,
            pltpu.SemaphoreType.DMA,
        ],
    )(tab, dst2d, src2d)


def _unpack_qkv(p):
    pb = lax.bitcast_convert_type(p, jnp.uint32)
    a = lax.bitcast_convert_type(pb & _HI, f32)
    b = lax.bitcast_convert_type(pb << 16, f32)
    q = a[:, :C]
    k = jnp.concatenate([a[:, C:96], b[:, :32]], axis=1)
    v = b[:, 32:96]
    return q, k, v


def _edge_body(g1_ref, g2_ref, ea_ref, wet_ref, wmut_ref, bmu_ref,
               lnag_ref, lnab_ref, wmlt_ref, bml_ref, lnmg_ref, lnmb_ref,
               m_ref):
    q_i, k_i, v_i = _unpack_qkv(g1_ref[...])
    _, k_j, v_j = _unpack_qkv(g2_ref[...])
    e = jnp.dot(ea_ref[...], wet_ref[...], preferred_element_type=f32)

    alpha = jnp.concatenate([q_i * k_i, q_i * k_j, q_i * e], axis=1)
    alpha = alpha * _SQRT3C_INV
    mu = jnp.mean(alpha, axis=1, keepdims=True)
    d = alpha - mu
    var = jnp.mean(d * d, axis=1, keepdims=True)
    ln = d * lax.rsqrt(var + _EPS) * lnag_ref[...] + lnab_ref[...]
    gate = jax.nn.sigmoid(ln)

    wmut = wmut_ref[...]
    u = (jnp.dot(v_i, wmut[:C], preferred_element_type=f32)
         + jnp.dot(v_j, wmut[C:2 * C], preferred_element_type=f32)
         + jnp.dot(e, wmut[2 * C:], preferred_element_type=f32)
         + bmu_ref[...])
    m = jnp.dot(u * gate, wmlt_ref[...], preferred_element_type=f32) + bml_ref[...]
    mmu = jnp.mean(m, axis=1, keepdims=True)
    md = m - mmu
    mvar = jnp.mean(md * md, axis=1, keepdims=True)
    mm = md * lax.rsqrt(mvar + _EPS) * lnmg_ref[...] + lnmb_ref[...]
    m_ref[...] = jnp.concatenate([mm, jnp.zeros_like(mm)], axis=1)


def _edge_math(g1, g2, ea, wet, wmut, bmu, lnag, lnab, wmlt, bml,
               lnmg, lnmb):
    BE = 2000
    grid = E // BE
    full = lambda r, c_: pl.BlockSpec((r, c_), lambda i: (0, 0))
    eblk = pl.BlockSpec((BE, 2 * C), lambda i: (i, 0))
    return pl.pallas_call(
        _edge_body,
        grid=(grid,),
        in_specs=[
            eblk,
            eblk,
            pl.BlockSpec((BE, 16), lambda i: (i, 0)),
            full(16, C),
            full(3 * C, 3 * C),
            full(1, 3 * C),
            full(1, 3 * C),
            full(1, 3 * C),
            full(3 * C, C),
            full(1, C),
            full(1, C),
            full(1, C),
        ],
        out_specs=pl.BlockSpec((BE, 2 * C), lambda i: (i, 0)),
        out_shape=jax.ShapeDtypeStruct((EPAD, 2 * C), f32),
    )(g1, g2, ea, wet, wmut, bmu, lnag, lnab, wmlt, bml, lnmg, lnmb)


def _scatter_body(m_hbm, dst2d_hbm, zeros_hbm, out_hbm, mrows_v, idx_v,
                  semm0, semm1, agg_sh):
    c = lax.axis_index("c")
    s = lax.axis_index("s")
    wid = s * 2 + c
    pltpu.sync_copy(zeros_hbm.at[pl.ds(s * RPT, RPT)],
                    agg_sh.at[pl.ds(s * RPT, RPT)])
    plsc.subcore_barrier()

    base_rows = wid * NCH
    pltpu.sync_copy(dst2d_hbm.at[pl.ds(base_rows, NCH)], idx_v)
    semm = (semm0, semm1)

    def load(j, b):
        pltpu.async_copy(m_hbm.at[pl.ds((base_rows + j) * CH, CH)],
                         mrows_v.at[b], semm[b])

    def drain_scatter(j, b):
        pltpu.make_async_copy(m_hbm.at[pl.ds((base_rows + j) * CH, CH)],
                              mrows_v.at[b], semm[b]).wait()
        pltpu.sync_copy(mrows_v.at[b], agg_sh.at[idx_v.at[j]], add=True)

    load(0, 0)

    def body(p, carry):
        j0 = 2 * p
        load(j0 + 1, 1)
        drain_scatter(j0, 0)

        @pl.when(j0 + 2 < NCH)
        def _():
            load(j0 + 2, 0)

        drain_scatter(j0 + 1, 1)
        return carry

    lax.fori_loop(0, NCH // 2, body, 0)
    plsc.subcore_barrier()
    pltpu.sync_copy(agg_sh.at[pl.ds(s * RPT, RPT)],
                    out_hbm.at[c].at[pl.ds(s * RPT, RPT)])


def _scatter(m, dst2d, zeros_np):
    mesh = plsc.VectorSubcoreMesh(core_axis_name="c", subcore_axis_name="s")
    return pl.kernel(
        _scatter_body,
        out_type=jax.ShapeDtypeStruct((2, NP, 2 * C), f32),
        mesh=mesh,
        scratch_types=[
            pltpu.VMEM((2, CH, 2 * C), f32),
            pltpu.VMEM((NCH, CH), jnp.int32),
            pltpu.SemaphoreType.DMA,
            pltpu.SemaphoreType.DMA,
            pltpu.VMEM_SHARED((NP, 2 * C), f32),
        ],
    )(m, dst2d, zeros_np)


def _out_body(agg_ref, x_ref, wcat_ref, bcat_ref, bng_ref, bnb_ref,
              wskip_ref, bskip_ref, out_ref):
    agg = (agg_ref[0, :N, :C] + agg_ref[1, :N, :C])
    o = jnp.dot(agg, wcat_ref[...], preferred_element_type=f32) + bcat_ref[...]
    mu = jnp.mean(o, axis=0, keepdims=True)
    d = o - mu
    var = jnp.mean(d * d, axis=0, keepdims=True)
    o = d * lax.rsqrt(var + _EPS) * bng_ref[...] + bnb_ref[...]
    o = o * jax.nn.sigmoid(o)
    skip = jnp.dot(x_ref[...], wskip_ref[...], preferred_element_type=f32)
    out_ref[...] = o + skip + bskip_ref[...]


def _node_out(agg2, x, wcat_t, bcat, bng, bnb, wskip_t, bskip):
    return pl.pallas_call(
        _out_body,
        out_shape=jax.ShapeDtypeStruct((N, C), f32),
    )(agg2, x, wcat_t, bcat, bng, bnb, wskip_t, bskip)


def kernel(x, edge_index, edge_attr, Wq, bq, Wk, bk, Wv, bv, We, W_mu, b_mu,
           ln_a_g, ln_a_b, W_ml, b_ml, ln_m_g, ln_m_b, W_cat, b_cat,
           bn_g, bn_b, W_skip, b_skip):
    src = edge_index[0].astype(jnp.int32)
    dst = edge_index[1].astype(jnp.int32)
    pad_e = EPAD - E
    src_pad = jnp.concatenate([src, jnp.full((pad_e,), N, jnp.int32)])
    dst_pad = jnp.concatenate([dst, jnp.full((pad_e,), N, jnp.int32)])
    dst2d = dst_pad.reshape(EPAD // CH, CH)
    src2d = src_pad.reshape(EPAD // CH, CH)

    wqkv_t = jnp.concatenate([Wq, Wk, Wv], axis=0).T
    bqkv = jnp.concatenate([bq, bk, bv]).reshape(1, 3 * C)

    tab = _node_proj(x, wqkv_t, bqkv)
    g1, g2 = _gather(tab, dst2d, src2d)

    m = _edge_math(
        g1, g2, edge_attr, We.T, W_mu.T, b_mu.reshape(1, -1),
        ln_a_g.reshape(1, -1), ln_a_b.reshape(1, -1), W_ml.T,
        b_ml.reshape(1, -1), ln_m_g.reshape(1, -1), ln_m_b.reshape(1, -1))

    zeros_np = jnp.zeros((NP, 2 * C), f32)
    agg2 = _scatter(m, dst2d, zeros_np)

    return _node_out(agg2, x, W_cat.T, b_cat.reshape(1, -1),
                     bn_g.reshape(1, -1), bn_b.reshape(1, -1), W_skip.T,
                     b_skip.reshape(1, -1))
